# baseline (speedup 1.0000x reference)
import functools

import numpy as np
import jax
import jax.numpy as jnp
from jax import lax
from jax.experimental import pallas as pl
from jax.experimental.pallas import tpu as pltpu
from jax.experimental.pallas import tpu_sc as plsc

_L = 16


def _sgn1(f):
    return (f > 64).astype(np.int32) + (f >= 64).astype(np.int32)


_ar14 = np.arange(16384, dtype=np.int32)
_LUT21 = (2 * _sgn1(_ar14 & 127) + 6 * _sgn1(_ar14 >> 7)).astype(np.int32)
_LUT3 = (18 * _sgn1(np.arange(128, dtype=np.int32))).astype(np.int32)


@functools.lru_cache(maxsize=None)
def _make_pack(N, nc, nworkers, chunk, R):
    mesh = plsc.VectorSubcoreMesh(core_axis_name="c", subcore_axis_name="s")

    nsub = chunk // R

    @functools.partial(
        pl.kernel, mesh=mesh,
        compiler_params=pltpu.CompilerParams(needs_layout_passes=False),
        out_type=(jax.ShapeDtypeStruct((N,), jnp.int32),
                  jax.ShapeDtypeStruct((N,), jnp.int32)),
        scratch_types=[pltpu.VMEM((R, 4), jnp.int32),
                       pltpu.VMEM((R, 4), jnp.int32),
                       pltpu.VMEM((chunk,), jnp.int32),
                       pltpu.VMEM((chunk,), jnp.int32),
                       pltpu.VMEM((chunk,), jnp.int32),
                       pltpu.SemaphoreType.DMA,
                       pltpu.SemaphoreType.DMA],
    )
    def pack(ref_hbm, qry_hbm, msk_hbm, rp_hbm, qp_hbm,
             buf_a, buf_b, mbuf, rout, qout, sem_a, sem_b):
        w = lax.axis_index("s") * nc + lax.axis_index("c")

        @pl.when(w < nworkers)
        def _():
            base = w * chunk
            pltpu.sync_copy(msk_hbm.at[pl.ds(base, chunk)], mbuf)
            lane = lax.iota(jnp.int32, _L)
            col1 = jnp.full((_L,), 1, jnp.int32)
            col2 = jnp.full((_L,), 2, jnp.int32)
            col3 = jnp.full((_L,), 3, jnp.int32)

            def cpin(hbm, j, buf, sem):
                return pltpu.make_async_copy(
                    hbm.at[pl.ds(base + j * R, R)], buf, sem)

            def one_table(hbm, emit):
                cpin(hbm, 0, buf_a, sem_a).start()
                cpin(hbm, 1, buf_b, sem_b).start()

                def stage(j, buf, sem, k):
                    cpin(hbm, j, buf, sem).wait()

                    def body(i, c2):
                        rows = lane + i * _L
                        emit(j * R + i * _L, rows, buf)
                        return c2

                    lax.fori_loop(0, R // _L, body, 0)

                    @pl.when(k < nsub // 2 - 1)
                    def _():
                        cpin(hbm, j + 2, buf, sem).start()

                def sweep(k, carry):
                    stage(2 * k, buf_a, sem_a, k)
                    stage(2 * k + 1, buf_b, sem_b, k)
                    return carry

                lax.fori_loop(0, nsub // 2, sweep, 0)

            def emit_ref(off, rows, buf):
                c1v = plsc.load_gather(buf, [rows, col1])
                c2v = plsc.load_gather(buf, [rows, col2])
                c3v = plsc.load_gather(buf, [rows, col3])
                rout[pl.ds(off, _L)] = ((c1v + 64) | ((c2v + 64) << 7)
                                        | ((c3v + 64) << 14))

            def emit_qry(off, rows, buf):
                q1 = plsc.load_gather(buf, [rows, col1])
                q2 = plsc.load_gather(buf, [rows, col2])
                q3 = plsc.load_gather(buf, [rows, col3])
                qout[pl.ds(off, _L)] = (q1 | (q2 << 7) | (q3 << 14)
                                        | (mbuf[pl.ds(off, _L)] << 21))

            one_table(ref_hbm, emit_ref)
            one_table(qry_hbm, emit_qry)
            pltpu.sync_copy(rout, rp_hbm.at[pl.ds(base, chunk)])
            pltpu.sync_copy(qout, qp_hbm.at[pl.ds(base, chunk)])

    return pack


@functools.lru_cache(maxsize=None)
def _make_main(E, N, nc, ew, B, UN):
    nb = ew // B
    ni = B // (_L * UN)
    mesh = plsc.VectorSubcoreMesh(core_axis_name="c", subcore_axis_name="s")

    @functools.partial(
        pl.kernel, mesh=mesh,
        compiler_params=pltpu.CompilerParams(needs_layout_passes=False),
        out_type=(jax.ShapeDtypeStruct((E,), jnp.int32),
                  jax.ShapeDtypeStruct((E,), jnp.int32)),
        scratch_types=[pltpu.VMEM((N,), jnp.int32),
                       pltpu.VMEM((16384,), jnp.int32),
                       pltpu.VMEM((128,), jnp.int32),
                       pltpu.VMEM((B,), jnp.int32),
                       pltpu.VMEM((B,), jnp.int32),
                       pltpu.VMEM((B,), jnp.int32),
                       pltpu.VMEM((B,), jnp.int32),
                       pltpu.VMEM((B,), jnp.int32),
                       pltpu.VMEM((B,), jnp.int32),
                       pltpu.SemaphoreType.DMA,
                       pltpu.SemaphoreType.DMA,
                       pltpu.SemaphoreType.DMA,
                       pltpu.SemaphoreType.DMA,
                       pltpu.SemaphoreType.DMA,
                       pltpu.SemaphoreType.DMA],
    )
    def main(rp_hbm, qp_hbm, eref_hbm, eqry_hbm, lut21_hbm, lut3_hbm,
             out_hbm, tmp_hbm, table, lut21, lut3,
             e_a, e_b, r_a, r_b, o_a, o_b,
             sin_a, sin_b, srn_a, srn_b, sout_a, sout_b):
        w = lax.axis_index("s") * nc + lax.axis_index("c")
        base = w * ew

        def gather_block(ebuf, vbuf):
            def gath(j, c2):
                jb = j * (_L * UN)
                for u in range(UN):
                    sl = pl.ds(jb + u * _L, _L)
                    vbuf[sl] = plsc.load_gather(table, [ebuf[sl]])
                return c2
            lax.fori_loop(0, ni, gath, 0)

        def compute_block(ebuf, rbuf, obuf):
            def comp(j, c2):
                jb = j * (_L * UN)
                for u in range(UN):
                    sl = pl.ds(jb + u * _L, _L)
                    qv = plsc.load_gather(table, [ebuf[sl]])
                    d = rbuf[sl] - (qv & 0x1FFFFF)
                    obuf[sl] = (plsc.load_gather(lut21, [d & 0x3FFF])
                                + plsc.load_gather(lut3, [d >> 14])
                                + (qv >> 21))
                return c2
            lax.fori_loop(0, ni, comp, 0)

        def cp(hbm, b, buf, sem):
            return pltpu.make_async_copy(hbm.at[pl.ds(base + b * B, B)],
                                         buf, sem)

        def cpo(buf, hbm, b, sem):
            return pltpu.make_async_copy(buf, hbm.at[pl.ds(base + b * B, B)],
                                         sem)

        pltpu.sync_copy(rp_hbm.at[pl.ds(0, N)], table)
        cp(eref_hbm, 0, e_a, sin_a).start()
        cp(eref_hbm, 1, e_b, sin_b).start()

        def stage_a(b, ebuf, vbuf, sin, sout, k):
            cp(eref_hbm, b, ebuf, sin).wait()

            @pl.when(k > 0)
            def _():
                cpo(vbuf, tmp_hbm, b - 2, sout).wait()

            gather_block(ebuf, vbuf)

            @pl.when(k < nb // 2 - 1)
            def _():
                cp(eref_hbm, b + 2, ebuf, sin).start()

            cpo(vbuf, tmp_hbm, b, sout).start()

        def body_a(k, carry):
            stage_a(2 * k, e_a, o_a, sin_a, sout_a, k)
            stage_a(2 * k + 1, e_b, o_b, sin_b, sout_b, k)
            return carry

        lax.fori_loop(0, nb // 2, body_a, 0)
        cpo(o_a, tmp_hbm, nb - 2, sout_a).wait()
        cpo(o_b, tmp_hbm, nb - 1, sout_b).wait()

        pltpu.sync_copy(qp_hbm.at[pl.ds(0, N)], table)
        pltpu.sync_copy(lut21_hbm, lut21)
        pltpu.sync_copy(lut3_hbm, lut3)
        cp(eqry_hbm, 0, e_a, sin_a).start()
        cp(tmp_hbm, 0, r_a, srn_a).start()
        cp(eqry_hbm, 1, e_b, sin_b).start()
        cp(tmp_hbm, 1, r_b, srn_b).start()

        def stage_b(b, ebuf, rbuf, obuf, sin, srn, sout, k):
            cp(eqry_hbm, b, ebuf, sin).wait()
            cp(tmp_hbm, b, rbuf, srn).wait()

            @pl.when(k > 0)
            def _():
                cpo(obuf, out_hbm, b - 2, sout).wait()

            compute_block(ebuf, rbuf, obuf)

            @pl.when(k < nb // 2 - 1)
            def _():
                cp(eqry_hbm, b + 2, ebuf, sin).start()
                cp(tmp_hbm, b + 2, rbuf, srn).start()

            cpo(obuf, out_hbm, b, sout).start()

        def body_b(k, carry):
            stage_b(2 * k, e_a, r_a, o_a, sin_a, srn_a, sout_a, k)
            stage_b(2 * k + 1, e_b, r_b, o_b, sin_b, srn_b, sout_b, k)
            return carry

        lax.fori_loop(0, nb // 2, body_b, 0)
        cpo(o_a, out_hbm, nb - 2, sout_a).wait()
        cpo(o_b, out_hbm, nb - 1, sout_b).wait()

    return main


def kernel(ref_bcoords, query_bcoords, volume_mask, e_ref, e_query):
    N = ref_bcoords.shape[0]
    E = e_ref.shape[0]
    info = plsc.get_sparse_core_info()
    nc, ns = info.num_cores, info.num_subcores
    nw = nc * ns

    for nworkers in range(nw, 0, -1):
        if N % nworkers == 0 and (N // nworkers) % _L == 0:
            break
    chunk = N // nworkers
    for R in (400, 200, 80, 16):
        if chunk % R == 0:
            break
    rp, qp = _make_pack(N, nc, nworkers, chunk, R)(
        ref_bcoords, query_bcoords, volume_mask)

    ew = E // nw
    for B, UN in ((2000, 25), (2000, 5), (1600, 5), (800, 5), (400, 5),
                  (80, 5), (16, 1)):
        if ew % B == 0 and (ew // B) % 2 == 0 and B % (_L * UN) == 0:
            break
    out, _ = _make_main(E, N, nc, ew, B, UN)(
        rp, qp, e_ref, e_query, jnp.asarray(_LUT21), jnp.asarray(_LUT3))
    return out


# speedup vs baseline: 1.2234x; 1.2234x over previous
import functools

import numpy as np
import jax
import jax.numpy as jnp
from jax import lax
from jax.experimental import pallas as pl
from jax.experimental.pallas import tpu as pltpu
from jax.experimental.pallas import tpu_sc as plsc

_L = 16


def _sgn1(f):
    return (f > 64).astype(np.int32) + (f >= 64).astype(np.int32)


_ar14 = np.arange(16384, dtype=np.int32)
_LUT21 = (2 * _sgn1(_ar14 & 127) + 6 * _sgn1(_ar14 >> 7)).astype(np.int32)
_LUT3 = (18 * _sgn1(np.arange(128, dtype=np.int32))).astype(np.int32)


@functools.lru_cache(maxsize=None)
def _make_pack(N, nc, nworkers, chunk, R):
    mesh = plsc.VectorSubcoreMesh(core_axis_name="c", subcore_axis_name="s")

    nsub = chunk // R

    @functools.partial(
        pl.kernel, mesh=mesh,
        compiler_params=pltpu.CompilerParams(needs_layout_passes=False),
        out_type=(jax.ShapeDtypeStruct((N,), jnp.int32),
                  jax.ShapeDtypeStruct((N,), jnp.int32)),
        scratch_types=[pltpu.VMEM((R, 4), jnp.int32),
                       pltpu.VMEM((R, 4), jnp.int32),
                       pltpu.VMEM((chunk,), jnp.int32),
                       pltpu.VMEM((chunk,), jnp.int32),
                       pltpu.VMEM((chunk,), jnp.int32),
                       pltpu.SemaphoreType.DMA,
                       pltpu.SemaphoreType.DMA],
    )
    def pack(ref_hbm, qry_hbm, msk_hbm, rp_hbm, qp_hbm,
             buf_a, buf_b, mbuf, rout, qout, sem_a, sem_b):
        w = lax.axis_index("s") * nc + lax.axis_index("c")

        @pl.when(w < nworkers)
        def _():
            base = w * chunk
            pltpu.sync_copy(msk_hbm.at[pl.ds(base, chunk)], mbuf)
            lane = lax.iota(jnp.int32, _L)
            col1 = jnp.full((_L,), 1, jnp.int32)
            col2 = jnp.full((_L,), 2, jnp.int32)
            col3 = jnp.full((_L,), 3, jnp.int32)

            def cpin(hbm, j, buf, sem):
                return pltpu.make_async_copy(
                    hbm.at[pl.ds(base + j * R, R)], buf, sem)

            def one_table(hbm, emit):
                cpin(hbm, 0, buf_a, sem_a).start()
                cpin(hbm, 1, buf_b, sem_b).start()

                def stage(j, buf, sem, k):
                    cpin(hbm, j, buf, sem).wait()

                    def body(i, c2):
                        rows = lane + i * _L
                        emit(j * R + i * _L, rows, buf)
                        return c2

                    lax.fori_loop(0, R // _L, body, 0)

                    @pl.when(k < nsub // 2 - 1)
                    def _():
                        cpin(hbm, j + 2, buf, sem).start()

                def sweep(k, carry):
                    stage(2 * k, buf_a, sem_a, k)
                    stage(2 * k + 1, buf_b, sem_b, k)
                    return carry

                lax.fori_loop(0, nsub // 2, sweep, 0)

            def emit_ref(off, rows, buf):
                c1v = plsc.load_gather(buf, [rows, col1])
                c2v = plsc.load_gather(buf, [rows, col2])
                c3v = plsc.load_gather(buf, [rows, col3])
                rout[pl.ds(off, _L)] = ((c1v + 64) | ((c2v + 64) << 7)
                                        | ((c3v + 64) << 14))

            def emit_qry(off, rows, buf):
                q1 = plsc.load_gather(buf, [rows, col1])
                q2 = plsc.load_gather(buf, [rows, col2])
                q3 = plsc.load_gather(buf, [rows, col3])
                qout[pl.ds(off, _L)] = (q1 | (q2 << 7) | (q3 << 14)
                                        | (mbuf[pl.ds(off, _L)] << 21))

            one_table(ref_hbm, emit_ref)
            one_table(qry_hbm, emit_qry)
            pltpu.sync_copy(rout, rp_hbm.at[pl.ds(base, chunk)])
            pltpu.sync_copy(qout, qp_hbm.at[pl.ds(base, chunk)])

    return pack


@functools.lru_cache(maxsize=None)
def _make_main(E, N, nc, ew, B, UN):
    nb = ew // B
    ni = B // (_L * UN)
    mesh = plsc.VectorSubcoreMesh(core_axis_name="c", subcore_axis_name="s")

    @functools.partial(
        pl.kernel, mesh=mesh,
        compiler_params=pltpu.CompilerParams(needs_layout_passes=False),
        out_type=(jax.ShapeDtypeStruct((E,), jnp.int32),
                  jax.ShapeDtypeStruct((E,), jnp.int32)),
        scratch_types=[pltpu.VMEM((N,), jnp.int32),
                       pltpu.VMEM((16384,), jnp.int32),
                       pltpu.VMEM((128,), jnp.int32),
                       pltpu.VMEM((B,), jnp.int32),
                       pltpu.VMEM((B,), jnp.int32),
                       pltpu.VMEM((B,), jnp.int32),
                       pltpu.VMEM((B,), jnp.int32),
                       pltpu.VMEM((B,), jnp.int32),
                       pltpu.VMEM((B,), jnp.int32),
                       pltpu.SemaphoreType.DMA,
                       pltpu.SemaphoreType.DMA,
                       pltpu.SemaphoreType.DMA,
                       pltpu.SemaphoreType.DMA,
                       pltpu.SemaphoreType.DMA,
                       pltpu.SemaphoreType.DMA],
    )
    def main(rp_hbm, qp_hbm, eref_hbm, eqry_hbm, lut21_hbm, lut3_hbm,
             out_hbm, tmp_hbm, table, lut21, lut3,
             e_a, e_b, r_a, r_b, o_a, o_b,
             sin_a, sin_b, srn_a, srn_b, sout_a, sout_b):
        w = lax.axis_index("s") * nc + lax.axis_index("c")
        base = w * ew

        def gather_block(ebuf, vbuf):
            def gath(j, c2):
                jb = j * (_L * UN)
                for u in range(UN):
                    sl = pl.ds(jb + u * _L, _L)
                    vbuf[sl] = plsc.load_gather(table, [ebuf[sl]])
                return c2
            lax.fori_loop(0, ni, gath, 0)

        def compute_block(ebuf, rbuf, obuf):
            def comp(j, c2):
                jb = j * (_L * UN)
                for u in range(UN):
                    sl = pl.ds(jb + u * _L, _L)
                    qv = plsc.load_gather(table, [ebuf[sl]])
                    d = rbuf[sl] - (qv & 0x1FFFFF)
                    obuf[sl] = (plsc.load_gather(lut21, [d & 0x3FFF])
                                + plsc.load_gather(lut3, [d >> 14])
                                + (qv >> 21))
                return c2
            lax.fori_loop(0, ni, comp, 0)

        def cp(hbm, b, buf, sem):
            return pltpu.make_async_copy(hbm.at[pl.ds(base + b * B, B)],
                                         buf, sem)

        def cpo(buf, hbm, b, sem):
            return pltpu.make_async_copy(buf, hbm.at[pl.ds(base + b * B, B)],
                                         sem)

        pltpu.sync_copy(rp_hbm.at[pl.ds(0, N)], table)
        cp(eref_hbm, 0, e_a, sin_a).start()
        cp(eref_hbm, 1, e_b, sin_b).start()

        def stage_a(b, ebuf, vbuf, sin, sout, k):
            cp(eref_hbm, b, ebuf, sin).wait()

            @pl.when(k > 0)
            def _():
                cpo(vbuf, tmp_hbm, b - 2, sout).wait()

            gather_block(ebuf, vbuf)

            @pl.when(k < nb // 2 - 1)
            def _():
                cp(eref_hbm, b + 2, ebuf, sin).start()

            cpo(vbuf, tmp_hbm, b, sout).start()

        def body_a(k, carry):
            stage_a(2 * k, e_a, o_a, sin_a, sout_a, k)
            stage_a(2 * k + 1, e_b, o_b, sin_b, sout_b, k)
            return carry

        lax.fori_loop(0, nb // 2, body_a, 0)
        cpo(o_a, tmp_hbm, nb - 2, sout_a).wait()
        cpo(o_b, tmp_hbm, nb - 1, sout_b).wait()

        pltpu.sync_copy(qp_hbm.at[pl.ds(0, N)], table)
        pltpu.sync_copy(lut21_hbm, lut21)
        pltpu.sync_copy(lut3_hbm, lut3)
        cp(eqry_hbm, 0, e_a, sin_a).start()
        cp(tmp_hbm, 0, r_a, srn_a).start()
        cp(eqry_hbm, 1, e_b, sin_b).start()
        cp(tmp_hbm, 1, r_b, srn_b).start()

        def stage_b(b, ebuf, rbuf, obuf, sin, srn, sout, k):
            cp(eqry_hbm, b, ebuf, sin).wait()
            cp(tmp_hbm, b, rbuf, srn).wait()

            @pl.when(k > 0)
            def _():
                cpo(obuf, out_hbm, b - 2, sout).wait()

            compute_block(ebuf, rbuf, obuf)

            @pl.when(k < nb // 2 - 1)
            def _():
                cp(eqry_hbm, b + 2, ebuf, sin).start()
                cp(tmp_hbm, b + 2, rbuf, srn).start()

            cpo(obuf, out_hbm, b, sout).start()

        def body_b(k, carry):
            stage_b(2 * k, e_a, r_a, o_a, sin_a, srn_a, sout_a, k)
            stage_b(2 * k + 1, e_b, r_b, o_b, sin_b, srn_b, sout_b, k)
            return carry

        lax.fori_loop(0, nb // 2, body_b, 0)
        cpo(o_a, out_hbm, nb - 2, sout_a).wait()
        cpo(o_b, out_hbm, nb - 1, sout_b).wait()

    return main


def kernel(ref_bcoords, query_bcoords, volume_mask, e_ref, e_query):
    N = ref_bcoords.shape[0]
    E = e_ref.shape[0]
    info = plsc.get_sparse_core_info()
    nc, ns = info.num_cores, info.num_subcores
    nw = nc * ns

    for nworkers in range(nw, 0, -1):
        if N % nworkers == 0 and (N // nworkers) % _L == 0:
            break
    chunk = N // nworkers
    for R in (400, 200, 80, 16):
        if chunk % R == 0:
            break
    rp, qp = _make_pack(N, nc, nworkers, chunk, R)(
        ref_bcoords, query_bcoords, volume_mask)

    ew = E // nw
    for B, UN in ((2000, 5), (1600, 5), (800, 5), (400, 5),
                  (80, 5), (16, 1)):
        if ew % B == 0 and (ew // B) % 2 == 0 and B % (_L * UN) == 0:
            break
    out, _ = _make_main(E, N, nc, ew, B, UN)(
        rp, qp, e_ref, e_query, jnp.asarray(_LUT21), jnp.asarray(_LUT3))
    return out


# speedup vs baseline: 1.8099x; 1.4793x over previous
import functools

import numpy as np
import jax
import jax.numpy as jnp
from jax import lax
from jax.experimental import pallas as pl
from jax.experimental.pallas import tpu as pltpu
from jax.experimental.pallas import tpu_sc as plsc

_L = 16


def _sgn1(f):
    return (f > 64).astype(np.int32) + (f >= 64).astype(np.int32)


_ar14 = np.arange(16384, dtype=np.int32)
_LUT21 = (2 * _sgn1(_ar14 & 127) + 6 * _sgn1(_ar14 >> 7)).astype(np.int32)
_LUT3 = (18 * _sgn1(np.arange(128, dtype=np.int32))).astype(np.int32)


@functools.lru_cache(maxsize=None)
def _make_pack(N, nc, nworkers, chunk, R):
    mesh = plsc.VectorSubcoreMesh(core_axis_name="c", subcore_axis_name="s")

    nsub = chunk // R

    @functools.partial(
        pl.kernel, mesh=mesh,
        compiler_params=pltpu.CompilerParams(needs_layout_passes=False),
        out_type=(jax.ShapeDtypeStruct((N,), jnp.int32),
                  jax.ShapeDtypeStruct((N,), jnp.int32)),
        scratch_types=[pltpu.VMEM((R, 4), jnp.int32),
                       pltpu.VMEM((R, 4), jnp.int32),
                       pltpu.VMEM((chunk,), jnp.int32),
                       pltpu.VMEM((chunk,), jnp.int32),
                       pltpu.VMEM((chunk,), jnp.int32),
                       pltpu.SemaphoreType.DMA,
                       pltpu.SemaphoreType.DMA],
    )
    def pack(ref_hbm, qry_hbm, msk_hbm, rp_hbm, qp_hbm,
             buf_a, buf_b, mbuf, rout, qout, sem_a, sem_b):
        w = lax.axis_index("s") * nc + lax.axis_index("c")

        @pl.when(w < nworkers)
        def _():
            base = w * chunk
            pltpu.sync_copy(msk_hbm.at[pl.ds(base, chunk)], mbuf)
            lane = lax.iota(jnp.int32, _L)
            col1 = jnp.full((_L,), 1, jnp.int32)
            col2 = jnp.full((_L,), 2, jnp.int32)
            col3 = jnp.full((_L,), 3, jnp.int32)

            def cpin(hbm, j, buf, sem):
                return pltpu.make_async_copy(
                    hbm.at[pl.ds(base + j * R, R)], buf, sem)

            def one_table(hbm, emit):
                cpin(hbm, 0, buf_a, sem_a).start()
                cpin(hbm, 1, buf_b, sem_b).start()

                def stage(j, buf, sem, k):
                    cpin(hbm, j, buf, sem).wait()

                    def body(i, c2):
                        rows = lane + i * _L
                        emit(j * R + i * _L, rows, buf)
                        return c2

                    lax.fori_loop(0, R // _L, body, 0)

                    @pl.when(k < nsub // 2 - 1)
                    def _():
                        cpin(hbm, j + 2, buf, sem).start()

                def sweep(k, carry):
                    stage(2 * k, buf_a, sem_a, k)
                    stage(2 * k + 1, buf_b, sem_b, k)
                    return carry

                lax.fori_loop(0, nsub // 2, sweep, 0)

            def emit_ref(off, rows, buf):
                c1v = plsc.load_gather(buf, [rows, col1])
                c2v = plsc.load_gather(buf, [rows, col2])
                c3v = plsc.load_gather(buf, [rows, col3])
                rout[pl.ds(off, _L)] = ((c1v + 64) | ((c2v + 64) << 7)
                                        | ((c3v + 64) << 14))

            def emit_qry(off, rows, buf):
                q1 = plsc.load_gather(buf, [rows, col1])
                q2 = plsc.load_gather(buf, [rows, col2])
                q3 = plsc.load_gather(buf, [rows, col3])
                qout[pl.ds(off, _L)] = (q1 | (q2 << 7) | (q3 << 14)
                                        | (mbuf[pl.ds(off, _L)] << 21))

            one_table(ref_hbm, emit_ref)
            one_table(qry_hbm, emit_qry)
            pltpu.sync_copy(rout, rp_hbm.at[pl.ds(base, chunk)])
            pltpu.sync_copy(qout, qp_hbm.at[pl.ds(base, chunk)])

    return pack


@functools.lru_cache(maxsize=None)
def _make_main(E, N, nc, ew, B, UN):
    nb = ew // B
    mesh = plsc.VectorSubcoreMesh(core_axis_name="c", subcore_axis_name="s")

    @functools.partial(
        pl.kernel, mesh=mesh,
        compiler_params=pltpu.CompilerParams(needs_layout_passes=False),
        out_type=(jax.ShapeDtypeStruct((E,), jnp.int32),
                  jax.ShapeDtypeStruct((E,), jnp.int32)),
        scratch_types=[pltpu.VMEM((N,), jnp.int32),
                       pltpu.VMEM((16384,), jnp.int32),
                       pltpu.VMEM((128,), jnp.int32),
                       pltpu.VMEM((B,), jnp.int32),
                       pltpu.VMEM((B,), jnp.int32),
                       pltpu.VMEM((B,), jnp.int32),
                       pltpu.VMEM((B,), jnp.int32),
                       pltpu.VMEM((B,), jnp.int32),
                       pltpu.VMEM((B,), jnp.int32),
                       pltpu.SemaphoreType.DMA,
                       pltpu.SemaphoreType.DMA,
                       pltpu.SemaphoreType.DMA,
                       pltpu.SemaphoreType.DMA,
                       pltpu.SemaphoreType.DMA,
                       pltpu.SemaphoreType.DMA],
    )
    def main(rp_hbm, qp_hbm, eref_hbm, eqry_hbm, lut21_hbm, lut3_hbm,
             out_hbm, tmp_hbm, table, lut21, lut3,
             e_a, e_b, r_a, r_b, o_a, o_b,
             sin_a, sin_b, srn_a, srn_b, sout_a, sout_b):
        w = lax.axis_index("s") * nc + lax.axis_index("c")
        base = w * ew

        def gather_block(ebuf, vbuf):
            @plsc.parallel_loop(0, B, step=_L, unroll=UN)
            def _(i):
                sl = pl.ds(i, _L)
                vbuf[sl] = plsc.load_gather(table, [ebuf[sl]])

        def compute_block(ebuf, rbuf, obuf):
            @plsc.parallel_loop(0, B, step=_L, unroll=UN)
            def _(i):
                sl = pl.ds(i, _L)
                qv = plsc.load_gather(table, [ebuf[sl]])
                d = rbuf[sl] - (qv & 0x1FFFFF)
                obuf[sl] = (plsc.load_gather(lut21, [d & 0x3FFF])
                            + plsc.load_gather(lut3, [d >> 14])
                            + (qv >> 21))

        def cp(hbm, b, buf, sem):
            return pltpu.make_async_copy(hbm.at[pl.ds(base + b * B, B)],
                                         buf, sem)

        def cpo(buf, hbm, b, sem):
            return pltpu.make_async_copy(buf, hbm.at[pl.ds(base + b * B, B)],
                                         sem)

        pltpu.sync_copy(rp_hbm.at[pl.ds(0, N)], table)
        cp(eref_hbm, 0, e_a, sin_a).start()
        cp(eref_hbm, 1, e_b, sin_b).start()

        def stage_a(b, ebuf, vbuf, sin, sout, k):
            cp(eref_hbm, b, ebuf, sin).wait()

            @pl.when(k > 0)
            def _():
                cpo(vbuf, tmp_hbm, b - 2, sout).wait()

            gather_block(ebuf, vbuf)

            @pl.when(k < nb // 2 - 1)
            def _():
                cp(eref_hbm, b + 2, ebuf, sin).start()

            cpo(vbuf, tmp_hbm, b, sout).start()

        def body_a(k, carry):
            stage_a(2 * k, e_a, o_a, sin_a, sout_a, k)
            stage_a(2 * k + 1, e_b, o_b, sin_b, sout_b, k)
            return carry

        lax.fori_loop(0, nb // 2, body_a, 0)
        cpo(o_a, tmp_hbm, nb - 2, sout_a).wait()
        cpo(o_b, tmp_hbm, nb - 1, sout_b).wait()

        pltpu.sync_copy(qp_hbm.at[pl.ds(0, N)], table)
        pltpu.sync_copy(lut21_hbm, lut21)
        pltpu.sync_copy(lut3_hbm, lut3)
        cp(eqry_hbm, 0, e_a, sin_a).start()
        cp(tmp_hbm, 0, r_a, srn_a).start()
        cp(eqry_hbm, 1, e_b, sin_b).start()
        cp(tmp_hbm, 1, r_b, srn_b).start()

        def stage_b(b, ebuf, rbuf, obuf, sin, srn, sout, k):
            cp(eqry_hbm, b, ebuf, sin).wait()
            cp(tmp_hbm, b, rbuf, srn).wait()

            @pl.when(k > 0)
            def _():
                cpo(obuf, out_hbm, b - 2, sout).wait()

            compute_block(ebuf, rbuf, obuf)

            @pl.when(k < nb // 2 - 1)
            def _():
                cp(eqry_hbm, b + 2, ebuf, sin).start()
                cp(tmp_hbm, b + 2, rbuf, srn).start()

            cpo(obuf, out_hbm, b, sout).start()

        def body_b(k, carry):
            stage_b(2 * k, e_a, r_a, o_a, sin_a, srn_a, sout_a, k)
            stage_b(2 * k + 1, e_b, r_b, o_b, sin_b, srn_b, sout_b, k)
            return carry

        lax.fori_loop(0, nb // 2, body_b, 0)
        cpo(o_a, out_hbm, nb - 2, sout_a).wait()
        cpo(o_b, out_hbm, nb - 1, sout_b).wait()

    return main


def kernel(ref_bcoords, query_bcoords, volume_mask, e_ref, e_query):
    N = ref_bcoords.shape[0]
    E = e_ref.shape[0]
    info = plsc.get_sparse_core_info()
    nc, ns = info.num_cores, info.num_subcores
    nw = nc * ns

    for nworkers in range(nw, 0, -1):
        if N % nworkers == 0 and (N // nworkers) % _L == 0:
            break
    chunk = N // nworkers
    for R in (400, 200, 80, 16):
        if chunk % R == 0:
            break
    rp, qp = _make_pack(N, nc, nworkers, chunk, R)(
        ref_bcoords, query_bcoords, volume_mask)

    ew = E // nw
    for B, UN in ((2000, 8), (1600, 8), (800, 8), (400, 4),
                  (80, 4), (16, 1)):
        if ew % B == 0 and (ew // B) % 2 == 0 and B % _L == 0:
            break
    out, _ = _make_main(E, N, nc, ew, B, UN)(
        rp, qp, e_ref, e_query, jnp.asarray(_LUT21), jnp.asarray(_LUT3))
    return out


# speedup vs baseline: 1.8128x; 1.0017x over previous
import functools

import numpy as np
import jax
import jax.numpy as jnp
from jax import lax
from jax.experimental import pallas as pl
from jax.experimental.pallas import tpu as pltpu
from jax.experimental.pallas import tpu_sc as plsc

_L = 16


def _sgn1(f):
    return (f > 64).astype(np.int32) + (f >= 64).astype(np.int32)


_ar14 = np.arange(16384, dtype=np.int32)
_LUT21 = (2 * _sgn1(_ar14 & 127) + 6 * _sgn1(_ar14 >> 7)).astype(np.int32)
_LUT3 = (18 * _sgn1(np.arange(128, dtype=np.int32))).astype(np.int32)


@functools.lru_cache(maxsize=None)
def _make_pack(N, nc, nworkers, chunk, R):
    mesh = plsc.VectorSubcoreMesh(core_axis_name="c", subcore_axis_name="s")

    nsub = chunk // R

    @functools.partial(
        pl.kernel, mesh=mesh,
        compiler_params=pltpu.CompilerParams(needs_layout_passes=False),
        out_type=(jax.ShapeDtypeStruct((N,), jnp.int32),
                  jax.ShapeDtypeStruct((N,), jnp.int32)),
        scratch_types=[pltpu.VMEM((R, 4), jnp.int32),
                       pltpu.VMEM((R, 4), jnp.int32),
                       pltpu.VMEM((chunk,), jnp.int32),
                       pltpu.VMEM((chunk,), jnp.int32),
                       pltpu.VMEM((chunk,), jnp.int32),
                       pltpu.SemaphoreType.DMA,
                       pltpu.SemaphoreType.DMA],
    )
    def pack(ref_hbm, qry_hbm, msk_hbm, rp_hbm, qp_hbm,
             buf_a, buf_b, mbuf, rout, qout, sem_a, sem_b):
        w = lax.axis_index("s") * nc + lax.axis_index("c")

        @pl.when(w < nworkers)
        def _():
            base = w * chunk
            pltpu.sync_copy(msk_hbm.at[pl.ds(base, chunk)], mbuf)
            lane = lax.iota(jnp.int32, _L)
            col1 = jnp.full((_L,), 1, jnp.int32)
            col2 = jnp.full((_L,), 2, jnp.int32)
            col3 = jnp.full((_L,), 3, jnp.int32)

            def cpin(hbm, j, buf, sem):
                return pltpu.make_async_copy(
                    hbm.at[pl.ds(base + j * R, R)], buf, sem)

            def one_table(hbm, emit):
                cpin(hbm, 0, buf_a, sem_a).start()
                cpin(hbm, 1, buf_b, sem_b).start()

                def stage(j, buf, sem, k):
                    cpin(hbm, j, buf, sem).wait()

                    @plsc.parallel_loop(0, R, step=_L, unroll=5)
                    def _(i):
                        rows = lane + i
                        emit(j * R + i, rows, buf)

                    @pl.when(k < nsub // 2 - 1)
                    def _():
                        cpin(hbm, j + 2, buf, sem).start()

                def sweep(k, carry):
                    stage(2 * k, buf_a, sem_a, k)
                    stage(2 * k + 1, buf_b, sem_b, k)
                    return carry

                lax.fori_loop(0, nsub // 2, sweep, 0)

            def emit_ref(off, rows, buf):
                c1v = plsc.load_gather(buf, [rows, col1])
                c2v = plsc.load_gather(buf, [rows, col2])
                c3v = plsc.load_gather(buf, [rows, col3])
                rout[pl.ds(off, _L)] = ((c1v + 64) | ((c2v + 64) << 7)
                                        | ((c3v + 64) << 14))

            def emit_qry(off, rows, buf):
                q1 = plsc.load_gather(buf, [rows, col1])
                q2 = plsc.load_gather(buf, [rows, col2])
                q3 = plsc.load_gather(buf, [rows, col3])
                qout[pl.ds(off, _L)] = (q1 | (q2 << 7) | (q3 << 14)
                                        | (mbuf[pl.ds(off, _L)] << 21))

            one_table(ref_hbm, emit_ref)
            one_table(qry_hbm, emit_qry)
            pltpu.sync_copy(rout, rp_hbm.at[pl.ds(base, chunk)])
            pltpu.sync_copy(qout, qp_hbm.at[pl.ds(base, chunk)])

    return pack


@functools.lru_cache(maxsize=None)
def _make_main(E, N, nc, ew, B, UN):
    nb = ew // B
    mesh = plsc.VectorSubcoreMesh(core_axis_name="c", subcore_axis_name="s")

    @functools.partial(
        pl.kernel, mesh=mesh,
        compiler_params=pltpu.CompilerParams(needs_layout_passes=False),
        out_type=(jax.ShapeDtypeStruct((E,), jnp.int32),
                  jax.ShapeDtypeStruct((E,), jnp.int32)),
        scratch_types=[pltpu.VMEM((N,), jnp.int32),
                       pltpu.VMEM((16384,), jnp.int32),
                       pltpu.VMEM((128,), jnp.int32),
                       pltpu.VMEM((B,), jnp.int32),
                       pltpu.VMEM((B,), jnp.int32),
                       pltpu.VMEM((B,), jnp.int32),
                       pltpu.VMEM((B,), jnp.int32),
                       pltpu.VMEM((B,), jnp.int32),
                       pltpu.VMEM((B,), jnp.int32),
                       pltpu.SemaphoreType.DMA,
                       pltpu.SemaphoreType.DMA,
                       pltpu.SemaphoreType.DMA,
                       pltpu.SemaphoreType.DMA,
                       pltpu.SemaphoreType.DMA,
                       pltpu.SemaphoreType.DMA],
    )
    def main(rp_hbm, qp_hbm, eref_hbm, eqry_hbm, lut21_hbm, lut3_hbm,
             out_hbm, tmp_hbm, table, lut21, lut3,
             e_a, e_b, r_a, r_b, o_a, o_b,
             sin_a, sin_b, srn_a, srn_b, sout_a, sout_b):
        w = lax.axis_index("s") * nc + lax.axis_index("c")
        base = w * ew

        def gather_block(ebuf, vbuf):
            @plsc.parallel_loop(0, B, step=_L, unroll=UN)
            def _(i):
                sl = pl.ds(i, _L)
                vbuf[sl] = plsc.load_gather(table, [ebuf[sl]])

        def compute_block(ebuf, rbuf, obuf):
            @plsc.parallel_loop(0, B, step=_L, unroll=UN)
            def _(i):
                sl = pl.ds(i, _L)
                qv = plsc.load_gather(table, [ebuf[sl]])
                d = rbuf[sl] - (qv & 0x1FFFFF)
                obuf[sl] = (plsc.load_gather(lut21, [d & 0x3FFF])
                            + plsc.load_gather(lut3, [d >> 14])
                            + (qv >> 21))

        def cp(hbm, b, buf, sem):
            return pltpu.make_async_copy(hbm.at[pl.ds(base + b * B, B)],
                                         buf, sem)

        def cpo(buf, hbm, b, sem):
            return pltpu.make_async_copy(buf, hbm.at[pl.ds(base + b * B, B)],
                                         sem)

        pltpu.sync_copy(rp_hbm.at[pl.ds(0, N)], table)
        cp(eref_hbm, 0, e_a, sin_a).start()
        cp(eref_hbm, 1, e_b, sin_b).start()

        def stage_a(b, ebuf, vbuf, sin, sout, k):
            cp(eref_hbm, b, ebuf, sin).wait()

            @pl.when(k > 0)
            def _():
                cpo(vbuf, tmp_hbm, b - 2, sout).wait()

            gather_block(ebuf, vbuf)

            @pl.when(k < nb // 2 - 1)
            def _():
                cp(eref_hbm, b + 2, ebuf, sin).start()

            cpo(vbuf, tmp_hbm, b, sout).start()

        def body_a(k, carry):
            stage_a(2 * k, e_a, o_a, sin_a, sout_a, k)
            stage_a(2 * k + 1, e_b, o_b, sin_b, sout_b, k)
            return carry

        lax.fori_loop(0, nb // 2, body_a, 0)
        cpo(o_a, tmp_hbm, nb - 2, sout_a).wait()
        cpo(o_b, tmp_hbm, nb - 1, sout_b).wait()

        pltpu.sync_copy(qp_hbm.at[pl.ds(0, N)], table)
        pltpu.sync_copy(lut21_hbm, lut21)
        pltpu.sync_copy(lut3_hbm, lut3)
        cp(eqry_hbm, 0, e_a, sin_a).start()
        cp(tmp_hbm, 0, r_a, srn_a).start()
        cp(eqry_hbm, 1, e_b, sin_b).start()
        cp(tmp_hbm, 1, r_b, srn_b).start()

        def stage_b(b, ebuf, rbuf, obuf, sin, srn, sout, k):
            cp(eqry_hbm, b, ebuf, sin).wait()
            cp(tmp_hbm, b, rbuf, srn).wait()

            @pl.when(k > 0)
            def _():
                cpo(obuf, out_hbm, b - 2, sout).wait()

            compute_block(ebuf, rbuf, obuf)

            @pl.when(k < nb // 2 - 1)
            def _():
                cp(eqry_hbm, b + 2, ebuf, sin).start()
                cp(tmp_hbm, b + 2, rbuf, srn).start()

            cpo(obuf, out_hbm, b, sout).start()

        def body_b(k, carry):
            stage_b(2 * k, e_a, r_a, o_a, sin_a, srn_a, sout_a, k)
            stage_b(2 * k + 1, e_b, r_b, o_b, sin_b, srn_b, sout_b, k)
            return carry

        lax.fori_loop(0, nb // 2, body_b, 0)
        cpo(o_a, out_hbm, nb - 2, sout_a).wait()
        cpo(o_b, out_hbm, nb - 1, sout_b).wait()

    return main


def kernel(ref_bcoords, query_bcoords, volume_mask, e_ref, e_query):
    N = ref_bcoords.shape[0]
    E = e_ref.shape[0]
    info = plsc.get_sparse_core_info()
    nc, ns = info.num_cores, info.num_subcores
    nw = nc * ns

    for nworkers in range(nw, 0, -1):
        if N % nworkers == 0 and (N // nworkers) % _L == 0:
            break
    chunk = N // nworkers
    for R in (400, 200, 80, 16):
        if chunk % R == 0:
            break
    rp, qp = _make_pack(N, nc, nworkers, chunk, R)(
        ref_bcoords, query_bcoords, volume_mask)

    ew = E // nw
    for B, UN in ((2000, 8), (1600, 8), (800, 8), (400, 4),
                  (80, 4), (16, 1)):
        if ew % B == 0 and (ew // B) % 2 == 0 and B % _L == 0:
            break
    out, _ = _make_main(E, N, nc, ew, B, UN)(
        rp, qp, e_ref, e_query, jnp.asarray(_LUT21), jnp.asarray(_LUT3))
    return out
